# baseline (speedup 1.0000x reference)
import jax
import jax.numpy as jnp
from jax import lax
from jax.experimental import pallas as pl
from jax.experimental.pallas import tpu as pltpu

B = 16
N = 10000
H = 128
F = 4
IN = 2 * H + 2 * F
G3 = 3 * H
PW = 3 * H
RW_U, RW_I = 0, G3
RF_U, RF_I = 2 * G3, 2 * G3 + B
RB = 2 * G3 + 2 * B


def _body(uid_ref, iid_ref, p_ref,
          ublk_ref, iblk_ref, umem_ref, imem_ref,
          nu_ref, ni_ref, uout_ref, iout_ref,
          ue_ref, ie_ref, xb_ref, xc_ref, sem_g):
    b = pl.program_id(0)

    @pl.when(b == 0)
    def _compute():
        gath = [pltpu.make_async_copy(umem_ref.at[k, uid_ref[k]], ue_ref.at[k],
                                      sem_g) for k in range(B)]
        gath += [pltpu.make_async_copy(imem_ref.at[k, iid_ref[k]], ie_ref.at[k],
                                       sem_g) for k in range(B)]
        for c in gath:
            c.start()
        for c in gath:
            c.wait()

        ue = ue_ref[...]
        ie = ie_ref[...]
        uf = p_ref[RF_U:RF_U + B, 0:F]
        itf = p_ref[RF_I:RF_I + B, 0:F]

        xb_ref[0, :, 0:F] = uf
        xb_ref[0, :, F:H] = ie[:, 0:H - F]
        xb_ref[1, :, 0:F] = itf
        xb_ref[1, :, F:H] = ue[:, 0:H - F]
        xc_ref[...] = jnp.zeros((2, B, H), jnp.float32)
        xc_ref[0, :, 0:F] = ie[:, H - F:H]
        xc_ref[0, :, F:2 * F] = itf
        xc_ref[1, :, 0:F] = ue[:, H - F:H]
        xc_ref[1, :, F:2 * F] = uf

        def matmul(x, w):
            return lax.dot_general(x, w, (((1,), (1,)), ((), ())),
                                   preferred_element_type=jnp.float32)

        def gru(e1, k, rw):
            g_a = matmul(e1, p_ref[rw:rw + G3, 0:H])
            g_b = matmul(xb_ref[k], p_ref[rw:rw + G3, H:2 * H])
            g_c = matmul(xc_ref[k], p_ref[rw:rw + G3, 2 * H:3 * H])
            bih = p_ref[RB + 2 * k:RB + 2 * k + 1, :]
            bhh = p_ref[RB + 2 * k + 1:RB + 2 * k + 2, :]
            gx = g_a + g_b + g_c + bih
            g = gx + bhh
            r = jax.nn.sigmoid(g[:, :H])
            z = jax.nn.sigmoid(g[:, H:2 * H])
            n = jnp.tanh(gx[:, 2 * H:] + r * bhh[:, 2 * H:])
            out = (1.0 - z) * n
            nrm = jnp.sqrt(jnp.sum(out * out, axis=1, keepdims=True))
            return out / jnp.maximum(nrm, 1e-12)

        nu_ref[...] = gru(ue, 0, RW_U)
        ni_ref[...] = gru(ie, 1, RW_I)

    uout_ref[...] = ublk_ref[...]
    iout_ref[...] = iblk_ref[...]

    uout_ref[0, pl.ds(uid_ref[b], 1), :] = nu_ref[pl.ds(b, 1), :]
    iout_ref[0, pl.ds(iid_ref[b], 1), :] = ni_ref[pl.ds(b, 1), :]


def kernel(user_ids, item_ids, user_features, item_features, user_memory,
           item_memory, W_ih_u, W_hh_u, b_ih_u, b_hh_u, W_ih_i, W_hh_i,
           b_ih_i, b_hh_i):
    del W_hh_u, W_hh_i
    packed = jnp.concatenate([
        jnp.pad(W_ih_u, ((0, 0), (0, PW - IN))),
        jnp.pad(W_ih_i, ((0, 0), (0, PW - IN))),
        jnp.pad(user_features, ((0, 0), (0, PW - F))),
        jnp.pad(item_features, ((0, 0), (0, PW - F))),
        b_ih_u[None, :], b_hh_u[None, :], b_ih_i[None, :], b_hh_i[None, :],
        jnp.zeros((4, PW), jnp.float32),
    ], axis=0)
    smem = pl.BlockSpec(memory_space=pltpu.SMEM)
    anym = pl.BlockSpec(memory_space=pltpu.MemorySpace.HBM)
    blk = pl.BlockSpec((1, N, H), lambda b: (b, 0, 0))
    f32 = jnp.float32
    return pl.pallas_call(
        _body,
        grid=(B,),
        out_shape=(
            jax.ShapeDtypeStruct((B, H), f32),
            jax.ShapeDtypeStruct((B, H), f32),
            jax.ShapeDtypeStruct((B, N, H), f32),
            jax.ShapeDtypeStruct((B, N, H), f32),
        ),
        in_specs=[smem, smem,
                  pl.BlockSpec((RB + 8, PW), lambda b: (0, 0)),
                  blk, blk, anym, anym],
        out_specs=(
            pl.BlockSpec((B, H), lambda b: (0, 0)),
            pl.BlockSpec((B, H), lambda b: (0, 0)),
            blk,
            blk,
        ),
        scratch_shapes=[
            pltpu.VMEM((B, H), f32),
            pltpu.VMEM((B, H), f32),
            pltpu.VMEM((2, B, H), f32),
            pltpu.VMEM((2, B, H), f32),
            pltpu.SemaphoreType.DMA,
        ],
    )(user_ids, item_ids, packed,
      user_memory, item_memory, user_memory, item_memory)


# speedup vs baseline: 1.0566x; 1.0566x over previous
import jax
import jax.numpy as jnp
from jax import lax
from jax.experimental import pallas as pl
from jax.experimental.pallas import tpu as pltpu

B = 16
N = 10000
H = 128
F = 4
IN = 2 * H + 2 * F
G3 = 3 * H
PW = 3 * H


def _body(uid_ref, iid_ref,
          uf_ref, itf_ref,
          p_ref,
          bihu_ref, bhhu_ref, bihi_ref, bhhi_ref,
          ublk_ref, iblk_ref, umem_ref, imem_ref,
          nu_ref, ni_ref, uout_ref, iout_ref,
          ue_ref, ie_ref, bias_ref, feat_ref, xb_ref, xc_ref, sem_g):
    b = pl.program_id(0)

    @pl.when(b == 0)
    def _compute():
        cps = [pltpu.make_async_copy(umem_ref.at[k, uid_ref[k]], ue_ref.at[k],
                                     sem_g) for k in range(B)]
        cps += [pltpu.make_async_copy(imem_ref.at[k, iid_ref[k]], ie_ref.at[k],
                                      sem_g) for k in range(B)]
        cps += [
            pltpu.make_async_copy(bihu_ref, bias_ref.at[0], sem_g),
            pltpu.make_async_copy(bhhu_ref, bias_ref.at[1], sem_g),
            pltpu.make_async_copy(bihi_ref, bias_ref.at[2], sem_g),
            pltpu.make_async_copy(bhhi_ref, bias_ref.at[3], sem_g),
            pltpu.make_async_copy(uf_ref, feat_ref.at[0], sem_g),
            pltpu.make_async_copy(itf_ref, feat_ref.at[1], sem_g),
        ]
        for c in cps:
            c.start()
        for c in cps:
            c.wait()

        ue = ue_ref[...]
        ie = ie_ref[...]
        uf = feat_ref[0]
        itf = feat_ref[1]

        xb_ref[0, :, 0:F] = uf
        xb_ref[0, :, F:H] = ie[:, 0:H - F]
        xb_ref[1, :, 0:F] = itf
        xb_ref[1, :, F:H] = ue[:, 0:H - F]
        xc_ref[...] = jnp.zeros((2, B, H), jnp.float32)
        xc_ref[0, :, 0:F] = ie[:, H - F:H]
        xc_ref[0, :, F:2 * F] = itf
        xc_ref[1, :, 0:F] = ue[:, H - F:H]
        xc_ref[1, :, F:2 * F] = uf

        def matmul(x, w):
            return lax.dot_general(x, w, (((1,), (1,)), ((), ())),
                                   preferred_element_type=jnp.float32)

        def gru(e1, k):
            rw = k * G3
            gx = (matmul(e1, p_ref[rw:rw + G3, 0:H])
                  + matmul(xb_ref[k], p_ref[rw:rw + G3, H:2 * H])
                  + matmul(xc_ref[k], p_ref[rw:rw + G3, 2 * H:3 * H])
                  + bias_ref[2 * k:2 * k + 1, :])
            bhh = bias_ref[2 * k + 1:2 * k + 2, :]
            g = gx + bhh
            r = jax.nn.sigmoid(g[:, :H])
            z = jax.nn.sigmoid(g[:, H:2 * H])
            n = jnp.tanh(gx[:, 2 * H:] + r * bhh[:, 2 * H:])
            out = (1.0 - z) * n
            nrm = jnp.sqrt(jnp.sum(out * out, axis=1, keepdims=True))
            return out / jnp.maximum(nrm, 1e-12)

        nu_ref[...] = gru(ue, 0)
        ni_ref[...] = gru(ie, 1)

    uout_ref[...] = ublk_ref[...]
    iout_ref[...] = iblk_ref[...]

    uout_ref[0, pl.ds(uid_ref[b], 1), :] = nu_ref[pl.ds(b, 1), :]
    iout_ref[0, pl.ds(iid_ref[b], 1), :] = ni_ref[pl.ds(b, 1), :]


def kernel(user_ids, item_ids, user_features, item_features, user_memory,
           item_memory, W_ih_u, W_hh_u, b_ih_u, b_hh_u, W_ih_i, W_hh_i,
           b_ih_i, b_hh_i):
    del W_hh_u, W_hh_i
    packed = jnp.concatenate([jnp.pad(W_ih_u, ((0, 0), (0, PW - IN))),
                              jnp.pad(W_ih_i, ((0, 4), (0, PW - IN)))], axis=0)
    smem = pl.BlockSpec(memory_space=pltpu.SMEM)
    anym = pl.BlockSpec(memory_space=pltpu.MemorySpace.HBM)
    blk = pl.BlockSpec((1, N, H), lambda b: (b, 0, 0))
    f32 = jnp.float32
    return pl.pallas_call(
        _body,
        grid=(B,),
        out_shape=(
            jax.ShapeDtypeStruct((B, H), f32),
            jax.ShapeDtypeStruct((B, H), f32),
            jax.ShapeDtypeStruct((B, N, H), f32),
            jax.ShapeDtypeStruct((B, N, H), f32),
        ),
        in_specs=[smem, smem, anym, anym,
                  pl.BlockSpec((2 * G3 + 4, PW), lambda b: (0, 0)),
                  anym, anym, anym, anym, blk, blk, anym, anym],
        out_specs=(
            pl.BlockSpec((B, H), lambda b: (0, 0)),
            pl.BlockSpec((B, H), lambda b: (0, 0)),
            blk,
            blk,
        ),
        scratch_shapes=[
            pltpu.VMEM((B, H), f32),
            pltpu.VMEM((B, H), f32),
            pltpu.VMEM((4, G3), f32),
            pltpu.VMEM((2, B, F), f32),
            pltpu.VMEM((2, B, H), f32),
            pltpu.VMEM((2, B, H), f32),
            pltpu.SemaphoreType.DMA,
        ],
    )(user_ids, item_ids, user_features, item_features, packed,
      b_ih_u, b_hh_u, b_ih_i, b_hh_i,
      user_memory, item_memory, user_memory, item_memory)


# speedup vs baseline: 1.0674x; 1.0102x over previous
import jax
import jax.numpy as jnp
from jax import lax
from jax.experimental import pallas as pl
from jax.experimental.pallas import tpu as pltpu

B = 16
N = 10000
H = 128
F = 4
IN = 2 * H + 2 * F
G3 = 3 * H
PW = 3 * H


def _body(uid_ref, iid_ref,
          p_ref,
          bihu_ref, bhhu_ref, bihi_ref, bhhi_ref,
          ublk_ref, iblk_ref, umem_ref, imem_ref,
          nu_ref, ni_ref, uout_ref, iout_ref,
          ue_ref, ie_ref, bias_ref, xb_ref, xc_ref, sem_g):
    b = pl.program_id(0)

    @pl.when(b == 0)
    def _compute():
        cps = [pltpu.make_async_copy(umem_ref.at[k, uid_ref[k]], ue_ref.at[k],
                                     sem_g) for k in range(B)]
        cps += [pltpu.make_async_copy(imem_ref.at[k, iid_ref[k]], ie_ref.at[k],
                                      sem_g) for k in range(B)]
        cps += [
            pltpu.make_async_copy(bihu_ref, bias_ref.at[0], sem_g),
            pltpu.make_async_copy(bhhu_ref, bias_ref.at[1], sem_g),
            pltpu.make_async_copy(bihi_ref, bias_ref.at[2], sem_g),
            pltpu.make_async_copy(bhhi_ref, bias_ref.at[3], sem_g),
        ]
        for c in cps:
            c.start()
        for c in cps:
            c.wait()

        ue = ue_ref[...]
        ie = ie_ref[...]
        uf = p_ref[2 * G3 + 4:2 * G3 + 4 + B, 0:F]
        itf = p_ref[2 * G3 + 4 + B:2 * G3 + 4 + 2 * B, 0:F]

        xb_ref[0, :, 0:F] = uf
        xb_ref[0, :, F:H] = ie[:, 0:H - F]
        xb_ref[1, :, 0:F] = itf
        xb_ref[1, :, F:H] = ue[:, 0:H - F]
        xc_ref[...] = jnp.zeros((2, B, H), jnp.float32)
        xc_ref[0, :, 0:F] = ie[:, H - F:H]
        xc_ref[0, :, F:2 * F] = itf
        xc_ref[1, :, 0:F] = ue[:, H - F:H]
        xc_ref[1, :, F:2 * F] = uf

        def matmul(x, w):
            return lax.dot_general(x, w, (((1,), (1,)), ((), ())),
                                   preferred_element_type=jnp.float32)

        def gru(e1, k):
            rw = k * G3
            gx = (matmul(e1, p_ref[rw:rw + G3, 0:H])
                  + matmul(xb_ref[k], p_ref[rw:rw + G3, H:2 * H])
                  + matmul(xc_ref[k], p_ref[rw:rw + G3, 2 * H:3 * H])
                  + bias_ref[2 * k:2 * k + 1, :])
            bhh = bias_ref[2 * k + 1:2 * k + 2, :]
            g = gx + bhh
            r = jax.nn.sigmoid(g[:, :H])
            z = jax.nn.sigmoid(g[:, H:2 * H])
            n = jnp.tanh(gx[:, 2 * H:] + r * bhh[:, 2 * H:])
            out = (1.0 - z) * n
            nrm = jnp.sqrt(jnp.sum(out * out, axis=1, keepdims=True))
            return out / jnp.maximum(nrm, 1e-12)

        nu_ref[...] = gru(ue, 0)
        ni_ref[...] = gru(ie, 1)

    uout_ref[...] = ublk_ref[...]
    iout_ref[...] = iblk_ref[...]

    uout_ref[0, pl.ds(uid_ref[b], 1), :] = nu_ref[pl.ds(b, 1), :]
    iout_ref[0, pl.ds(iid_ref[b], 1), :] = ni_ref[pl.ds(b, 1), :]


def kernel(user_ids, item_ids, user_features, item_features, user_memory,
           item_memory, W_ih_u, W_hh_u, b_ih_u, b_hh_u, W_ih_i, W_hh_i,
           b_ih_i, b_hh_i):
    del W_hh_u, W_hh_i
    packed = jnp.concatenate([jnp.pad(W_ih_u, ((0, 0), (0, PW - IN))),
                              jnp.pad(W_ih_i, ((0, 4), (0, PW - IN))),
                              jnp.pad(user_features, ((0, 0), (0, PW - F))),
                              jnp.pad(item_features, ((0, 4), (0, PW - F)))],
                             axis=0)
    smem = pl.BlockSpec(memory_space=pltpu.SMEM)
    anym = pl.BlockSpec(memory_space=pltpu.MemorySpace.HBM)
    blk = pl.BlockSpec((1, N, H), lambda b: (b, 0, 0))
    f32 = jnp.float32
    return pl.pallas_call(
        _body,
        grid=(B,),
        out_shape=(
            jax.ShapeDtypeStruct((B, H), f32),
            jax.ShapeDtypeStruct((B, H), f32),
            jax.ShapeDtypeStruct((B, N, H), f32),
            jax.ShapeDtypeStruct((B, N, H), f32),
        ),
        in_specs=[smem, smem,
                  pl.BlockSpec((2 * G3 + 4 + 2 * B + 4, PW), lambda b: (0, 0)),
                  anym, anym, anym, anym, blk, blk, anym, anym],
        out_specs=(
            pl.BlockSpec((B, H), lambda b: (0, 0)),
            pl.BlockSpec((B, H), lambda b: (0, 0)),
            blk,
            blk,
        ),
        scratch_shapes=[
            pltpu.VMEM((B, H), f32),
            pltpu.VMEM((B, H), f32),
            pltpu.VMEM((4, G3), f32),
            pltpu.VMEM((2, B, H), f32),
            pltpu.VMEM((2, B, H), f32),
            pltpu.SemaphoreType.DMA,
        ],
    )(user_ids, item_ids, packed,
      b_ih_u, b_hh_u, b_ih_i, b_hh_i,
      user_memory, item_memory, user_memory, item_memory)
